# baseline (speedup 1.0000x reference)
import functools

import jax
import jax.numpy as jnp
from jax import lax
from jax.experimental import pallas as pl
from jax.experimental.pallas import tpu as pltpu
from jax.experimental.pallas import tpu_sc as plsc

B = 1024
HIST = 50
GH = 50
GA = 24
GB = 24
GC = 2
HP = 64
D = 1024
NPH = 582

NC = 2
NS = 16
L = 16
NW = NC * NS
BPW = B // NW


def _sc_pool(xp, table, nb_rows):
    mesh = plsc.VectorSubcoreMesh(core_axis_name="c", subcore_axis_name="s")
    bpw = nb_rows // NW

    @functools.partial(
        pl.kernel,
        mesh=mesh,
        out_type=jax.ShapeDtypeStruct((nb_rows, D), jnp.float32),
        scratch_types=[
            pltpu.VMEM((bpw, HP), jnp.int32),
            pltpu.VMEM((2, D), jnp.float32),
            pltpu.SemaphoreType.DMA,
            pltpu.SemaphoreType.DMA,
            pltpu.SemaphoreType.DMA,
            pltpu.SemaphoreType.DMA,
        ],
    )
    def pool(x_hbm, table_hbm, out_hbm, idx_v, outb, g0, g1, o0, o1):
        wid = lax.axis_index("s") * NC + lax.axis_index("c")
        base = wid * bpw
        pltpu.sync_copy(x_hbm.at[pl.ds(base, bpw)], idx_v)

        def body(bufa0, bufa1, bufb0, bufb1, bufc0, bufc1):
            bufas = (bufa0, bufa1)
            bufbs = (bufb0, bufb1)
            bufcs = (bufc0, bufc1)
            gsems = (g0, g1)
            osems = (o0, o1)

            def issue_gather(i, nb):
                pltpu.async_copy(
                    table_hbm.at[idx_v.at[i, pl.ds(0, GA)]], bufas[nb],
                    gsems[nb])
                pltpu.async_copy(
                    table_hbm.at[idx_v.at[i, pl.ds(GA, GB)]], bufbs[nb],
                    gsems[nb])
                pltpu.async_copy(
                    table_hbm.at[idx_v.at[i, pl.ds(GA + GB, GC)]],
                    bufcs[nb], gsems[nb])

            def wait_gather(nb):
                pltpu.make_async_copy(
                    table_hbm.at[idx_v.at[0, pl.ds(0, GA)]], bufas[nb],
                    gsems[nb]).wait()
                pltpu.make_async_copy(
                    table_hbm.at[idx_v.at[0, pl.ds(GA, GB)]], bufbs[nb],
                    gsems[nb]).wait()
                pltpu.make_async_copy(
                    table_hbm.at[idx_v.at[0, pl.ds(GA + GB, GC)]],
                    bufcs[nb], gsems[nb]).wait()

            issue_gather(0, 0)
            issue_gather(1, 1)

            def row(i, nb):
                bufa = bufas[nb]
                bufb = bufbs[nb]
                bufc = bufcs[nb]
                wait_gather(nb)

                @pl.when(i >= 2)
                def _():
                    pltpu.make_async_copy(
                        outb.at[pl.ds(nb, 1)], out_hbm.at[pl.ds(base, 1)],
                        osems[nb]).wait()

                def chunk_body(v, carry):
                    o = pl.multiple_of(v * L, L)
                    accs = [bufa[j, pl.ds(o, L)] for j in range(4)]
                    for j in range(4, GA):
                        accs[j % 4] = accs[j % 4] + bufa[j, pl.ds(o, L)]
                    for j in range(GB):
                        accs[j % 4] = accs[j % 4] + bufb[j, pl.ds(o, L)]
                    for j in range(GC):
                        accs[j % 4] = accs[j % 4] + bufc[j, pl.ds(o, L)]
                    outb[nb, pl.ds(o, L)] = (
                        (accs[0] + accs[1]) + (accs[2] + accs[3]))
                    return carry

                lax.fori_loop(0, D // L, chunk_body, 0)

                pltpu.async_copy(
                    outb.at[pl.ds(nb, 1)], out_hbm.at[pl.ds(base + i, 1)],
                    osems[nb])

                @pl.when(i + 2 < bpw)
                def _():
                    issue_gather(i + 2, nb)

            def step(s, carry):
                row(2 * s, 0)
                row(2 * s + 1, 1)
                return carry

            lax.fori_loop(0, bpw // 2, step, 0)

            for nb in range(2):
                pltpu.make_async_copy(
                    outb.at[pl.ds(nb, 1)], out_hbm.at[pl.ds(base, 1)],
                    osems[nb]).wait()

        pl.run_scoped(
            body,
            pltpu.VMEM((GA, D), jnp.float32),
            pltpu.VMEM((GA, D), jnp.float32),
            pltpu.VMEM((GB, D), jnp.float32),
            pltpu.VMEM((GB, D), jnp.float32),
            pltpu.VMEM((GC, D), jnp.float32),
            pltpu.VMEM((GC, D), jnp.float32),
        )

    return pool(xp, table)


def _tc_head(pooled, x, t0row, W, b2, nb_rows):
    TB = 256

    def body(p_ref, x_ref, t0_ref, w_ref, b_ref, o_ref):
        n0 = jnp.sum((x_ref[...] == 0).astype(jnp.float32), axis=1,
                     keepdims=True)
        vr = jnp.tanh(p_ref[...] - (n0 + float(GH - HIST)) * t0_ref[...])
        logits = jnp.dot(vr, w_ref[...],
                         preferred_element_type=jnp.float32) + b_ref[...]
        m = jnp.max(logits, axis=-1, keepdims=True)
        e = jnp.exp(logits - m)
        o_ref[...] = e / jnp.sum(e, axis=-1, keepdims=True)

    return pl.pallas_call(
        body,
        grid=(nb_rows // TB,),
        in_specs=[
            pl.BlockSpec((TB, D), lambda i: (i, 0)),
            pl.BlockSpec((TB, HIST), lambda i: (i, 0)),
            pl.BlockSpec((1, D), lambda i: (0, 0)),
            pl.BlockSpec((D, NPH), lambda i: (0, 0)),
            pl.BlockSpec((1, NPH), lambda i: (0, 0)),
        ],
        out_specs=pl.BlockSpec((TB, NPH), lambda i: (i, 0)),
        out_shape=jax.ShapeDtypeStruct((nb_rows, NPH), jnp.float32),
    )(pooled, x, t0row, W, b2)


def kernel(x, table, W, b):
    x = x.astype(jnp.int32)
    xp = jnp.pad(x, ((0, 0), (0, HP - HIST)), constant_values=0)
    t0 = table[0:1]
    b2 = b.reshape(1, NPH)
    half = B // 2
    outs = []
    for h in range(2):
        sl = slice(h * half, (h + 1) * half)
        pooled = _sc_pool(xp[sl], table, half)
        outs.append(_tc_head(pooled, x[sl], t0, W, b2, half))
    return jnp.concatenate(outs, axis=0)


# speedup vs baseline: 1.0836x; 1.0836x over previous
import functools

import jax
import jax.numpy as jnp
from jax import lax
from jax.experimental import pallas as pl
from jax.experimental.pallas import tpu as pltpu
from jax.experimental.pallas import tpu_sc as plsc

B = 1024
HIST = 50
GH = 50
GA = 24
GB = 24
GC = 2
HP = 64
D = 1024
NPH = 582

NC = 2
NS = 16
L = 16
NW = NC * NS
BPW = B // NW


def _sc_pool(xp, table, nb_rows):
    mesh = plsc.VectorSubcoreMesh(core_axis_name="c", subcore_axis_name="s")
    bpw = nb_rows // NW

    @functools.partial(
        pl.kernel,
        mesh=mesh,
        out_type=jax.ShapeDtypeStruct((nb_rows, D), jnp.float32),
        scratch_types=[
            pltpu.VMEM((bpw, HP), jnp.int32),
            pltpu.VMEM((2, D), jnp.float32),
            pltpu.SemaphoreType.DMA,
            pltpu.SemaphoreType.DMA,
            pltpu.SemaphoreType.DMA,
            pltpu.SemaphoreType.DMA,
        ],
    )
    def pool(x_hbm, table_hbm, out_hbm, idx_v, outb, g0, g1, o0, o1):
        wid = lax.axis_index("s") * NC + lax.axis_index("c")
        base = wid * bpw
        pltpu.sync_copy(x_hbm.at[pl.ds(base, bpw)], idx_v)

        def body(bufa0, bufa1, bufb0, bufb1, bufc0, bufc1):
            bufas = (bufa0, bufa1)
            bufbs = (bufb0, bufb1)
            bufcs = (bufc0, bufc1)
            gsems = (g0, g1)
            osems = (o0, o1)

            def issue_gather(i, nb):
                pltpu.async_copy(
                    table_hbm.at[idx_v.at[i, pl.ds(0, GA)]], bufas[nb],
                    gsems[nb])
                pltpu.async_copy(
                    table_hbm.at[idx_v.at[i, pl.ds(GA, GB)]], bufbs[nb],
                    gsems[nb])
                pltpu.async_copy(
                    table_hbm.at[idx_v.at[i, pl.ds(GA + GB, GC)]],
                    bufcs[nb], gsems[nb])

            def wait_gather(nb):
                pltpu.make_async_copy(
                    table_hbm.at[idx_v.at[0, pl.ds(0, GA)]], bufas[nb],
                    gsems[nb]).wait()
                pltpu.make_async_copy(
                    table_hbm.at[idx_v.at[0, pl.ds(GA, GB)]], bufbs[nb],
                    gsems[nb]).wait()
                pltpu.make_async_copy(
                    table_hbm.at[idx_v.at[0, pl.ds(GA + GB, GC)]],
                    bufcs[nb], gsems[nb]).wait()

            issue_gather(0, 0)
            issue_gather(1, 1)

            def row(i, nb):
                bufa = bufas[nb]
                bufb = bufbs[nb]
                bufc = bufcs[nb]
                wait_gather(nb)

                @pl.when(i >= 2)
                def _():
                    pltpu.make_async_copy(
                        outb.at[pl.ds(nb, 1)], out_hbm.at[pl.ds(base, 1)],
                        osems[nb]).wait()

                def chunk_body(v, carry):
                    o = pl.multiple_of(v * L, L)
                    accs = [bufa[j, pl.ds(o, L)] for j in range(4)]
                    for j in range(4, GA):
                        accs[j % 4] = accs[j % 4] + bufa[j, pl.ds(o, L)]
                    for j in range(GB):
                        accs[j % 4] = accs[j % 4] + bufb[j, pl.ds(o, L)]
                    for j in range(GC):
                        accs[j % 4] = accs[j % 4] + bufc[j, pl.ds(o, L)]
                    outb[nb, pl.ds(o, L)] = (
                        (accs[0] + accs[1]) + (accs[2] + accs[3]))
                    return carry

                lax.fori_loop(0, D // L, chunk_body, 0)

                pltpu.async_copy(
                    outb.at[pl.ds(nb, 1)], out_hbm.at[pl.ds(base + i, 1)],
                    osems[nb])

                @pl.when(i + 2 < bpw)
                def _():
                    issue_gather(i + 2, nb)

            def step(s, carry):
                row(2 * s, 0)
                row(2 * s + 1, 1)
                return carry

            lax.fori_loop(0, bpw // 2, step, 0)

            for nb in range(2):
                pltpu.make_async_copy(
                    outb.at[pl.ds(nb, 1)], out_hbm.at[pl.ds(base, 1)],
                    osems[nb]).wait()

        pl.run_scoped(
            body,
            pltpu.VMEM((GA, D), jnp.float32),
            pltpu.VMEM((GA, D), jnp.float32),
            pltpu.VMEM((GB, D), jnp.float32),
            pltpu.VMEM((GB, D), jnp.float32),
            pltpu.VMEM((GC, D), jnp.float32),
            pltpu.VMEM((GC, D), jnp.float32),
        )

    return pool(xp, table)


def _tc_head(pooled, x, t0row, W, b2, nb_rows):
    TB = 256

    def body(p_ref, x_ref, t0_ref, w_ref, b_ref, o_ref):
        n0 = jnp.sum((x_ref[...] == 0).astype(jnp.float32), axis=1,
                     keepdims=True)
        vr = jnp.tanh(p_ref[...] - (n0 + float(GH - HIST)) * t0_ref[...])
        logits = jnp.dot(vr, w_ref[...],
                         preferred_element_type=jnp.float32) + b_ref[...]
        m = jnp.max(logits, axis=-1, keepdims=True)
        e = jnp.exp(logits - m)
        o_ref[...] = e / jnp.sum(e, axis=-1, keepdims=True)

    return pl.pallas_call(
        body,
        grid=(nb_rows // TB,),
        in_specs=[
            pl.BlockSpec((TB, D), lambda i: (i, 0)),
            pl.BlockSpec((TB, HIST), lambda i: (i, 0)),
            pl.BlockSpec((1, D), lambda i: (0, 0)),
            pl.BlockSpec((D, NPH), lambda i: (0, 0)),
            pl.BlockSpec((1, NPH), lambda i: (0, 0)),
        ],
        out_specs=pl.BlockSpec((TB, NPH), lambda i: (i, 0)),
        out_shape=jax.ShapeDtypeStruct((nb_rows, NPH), jnp.float32),
    )(pooled, x, t0row, W, b2)


def kernel(x, table, W, b):
    x = x.astype(jnp.int32)
    xp = jnp.pad(x, ((0, 0), (0, HP - HIST)), constant_values=0)
    pooled = _sc_pool(xp, table, B)
    return _tc_head(pooled, x, table[0:1], W, b.reshape(1, NPH), B)
